# baseline (speedup 1.0000x reference)
import functools

import jax
import jax.numpy as jnp
from jax import lax
from jax.experimental import pallas as pl
from jax.experimental.pallas import tpu as pltpu
from jax.experimental.pallas import tpu_sc as plsc

RESOLUTIONS = (16, 32, 64, 128, 256, 512, 1024, 2048)
N_SMALL = 5
SMALL_OFF = (0, 256, 1280, 5376, 21760)
SMALL_TOT = 87296
BN = 524288
NC, NS = 2, 16
NW = NC * NS
NPT = BN // NW
C = 1024
NCH = NPT // C
SL = C // 16

_CLIP_HI = 1.0 - 1e-6


def _cell(xc, yc, r):
    rf = jnp.float32(r - 1)
    xs = xc * rf
    ys = yc * rf
    x0 = xs.astype(jnp.int32)
    y0 = ys.astype(jnp.int32)
    fx = xs - x0.astype(jnp.float32)
    fy = ys - y0.astype(jnp.float32)
    idx = y0 * r + x0
    return idx, fx, fy


def _lerp_packed(w00, w10, w01, w11, fx, fy):
    c00 = plsc.bitcast(w00, jnp.bfloat16)
    c10 = plsc.bitcast(w10, jnp.bfloat16)
    c01 = plsc.bitcast(w01, jnp.bfloat16)
    c11 = plsc.bitcast(w11, jnp.bfloat16)
    fxd = plsc.pack(fx, fx, format=plsc.PackFormat.INTERLEAVED)
    fyd = plsc.pack(fy, fy, format=plsc.PackFormat.INTERLEAVED)
    l0 = c00 + (c10 - c00) * fxd
    l1 = c01 + (c11 - c01) * fxd
    lf = l0 + (l1 - l0) * fyd
    f0, f1 = plsc.unpack(lf, format=plsc.PackFormat.INTERLEAVED)
    return f0, f1


@functools.partial(
    pl.kernel,
    out_type=(jax.ShapeDtypeStruct((BN,), jnp.float32),
              jax.ShapeDtypeStruct((BN,), jnp.float32)),
    mesh=plsc.VectorSubcoreMesh(core_axis_name="c", subcore_axis_name="s",
                                num_cores=NC, num_subcores=NS),
    scratch_types=[
        pltpu.VMEM((SMALL_TOT,), jnp.int32),
        pltpu.VMEM((C,), jnp.float32),
        pltpu.VMEM((C,), jnp.float32),
        pltpu.VMEM((C,), jnp.float32),
        pltpu.VMEM((C,), jnp.float32),
        pltpu.VMEM((2 * C,), jnp.float32),
        pltpu.VMEM((2 * C,), jnp.float32),
        pltpu.VMEM((2 * C,), jnp.float32),
        pltpu.VMEM((4 * C,), jnp.int32),
        pltpu.VMEM((4 * C,), jnp.int32),
        pltpu.VMEM((4 * C,), jnp.int32),
        pltpu.VMEM((4 * C,), jnp.int32),
        pltpu.VMEM((4 * C,), jnp.int32),
        pltpu.VMEM((4 * C,), jnp.int32),
        pltpu.SemaphoreType.DMA,
        pltpu.SemaphoreType.DMA,
        pltpu.SemaphoreType.DMA,
    ],
    compiler_params=pltpu.CompilerParams(needs_layout_passes=False),
)
def _grid_kernel(x_hbm, y_hbm, small_hbm, t5_hbm, t6_hbm, t7_hbm,
                 out0_hbm, out1_hbm,
                 small_v, xv, yv, a0v, a1v, fw5, fw6, fw7,
                 idx5, idx6, idx7, rows5, rows6, rows7,
                 sem5, sem6, sem7):
    wid = lax.axis_index("s") * NC + lax.axis_index("c")
    base = wid * NPT
    pltpu.sync_copy(small_hbm, small_v)
    sems = (sem5, sem6, sem7)
    tabs = (t5_hbm, t6_hbm, t7_hbm)
    fws = (fw5, fw6, fw7)
    idxs = (idx5, idx6, idx7)
    rows = (rows5, rows6, rows7)
    for g in range(NCH):
        cbase = base + g * C
        pltpu.sync_copy(x_hbm.at[pl.ds(cbase, C)], xv)
        pltpu.sync_copy(y_hbm.at[pl.ds(cbase, C)], yv)

        def idx_body(s, carry):
            o = s * 16
            xc = jnp.clip(xv[pl.ds(o, 16)], 0.0, _CLIP_HI)
            yc = jnp.clip(yv[pl.ds(o, 16)], 0.0, _CLIP_HI)
            for li, r in enumerate(RESOLUTIONS[N_SMALL:]):
                idx, fx, fy = _cell(xc, yc, r)
                iv = idxs[li]
                iv[pl.ds(o, 16)] = idx
                iv[pl.ds(C + o, 16)] = idx + 1
                iv[pl.ds(2 * C + o, 16)] = idx + r
                iv[pl.ds(3 * C + o, 16)] = idx + r + 1
                fws[li][pl.ds(o, 16)] = fx
                fws[li][pl.ds(C + o, 16)] = fy
            return carry

        lax.fori_loop(0, SL, idx_body, 0)

        copies = [
            pltpu.async_copy(tabs[li].at[idxs[li]], rows[li], sems[li])
            for li in range(3)
        ]

        def small_body(s, carry):
            o = s * 16
            xc = jnp.clip(xv[pl.ds(o, 16)], 0.0, _CLIP_HI)
            yc = jnp.clip(yv[pl.ds(o, 16)], 0.0, _CLIP_HI)
            a0 = jnp.zeros((16,), jnp.float32)
            a1 = jnp.zeros((16,), jnp.float32)
            for l in range(N_SMALL):
                r = RESOLUTIONS[l]
                idx, fx, fy = _cell(xc, yc, r)
                b = idx + SMALL_OFF[l]
                w00 = plsc.load_gather(small_v, [b])
                w10 = plsc.load_gather(small_v, [b + 1])
                w01 = plsc.load_gather(small_v, [b + r])
                w11 = plsc.load_gather(small_v, [b + r + 1])
                f0, f1 = _lerp_packed(w00, w10, w01, w11, fx, fy)
                a0 = a0 + f0
                a1 = a1 + f1
            a0v[pl.ds(o, 16)] = a0
            a1v[pl.ds(o, 16)] = a1
            return carry

        lax.fori_loop(0, SL, small_body, 0)

        for cp in copies:
            cp.wait()

        def acc_body(s, carry):
            o = s * 16
            a0 = a0v[pl.ds(o, 16)]
            a1 = a1v[pl.ds(o, 16)]
            for li in range(3):
                rv = rows[li]
                w00 = rv[pl.ds(o, 16)]
                w10 = rv[pl.ds(C + o, 16)]
                w01 = rv[pl.ds(2 * C + o, 16)]
                w11 = rv[pl.ds(3 * C + o, 16)]
                f0, f1 = _lerp_packed(w00, w10, w01, w11,
                                      fws[li][pl.ds(o, 16)],
                                      fws[li][pl.ds(C + o, 16)])
                a0 = a0 + f0
                a1 = a1 + f1
            a0v[pl.ds(o, 16)] = a0
            a1v[pl.ds(o, 16)] = a1
            return carry

        lax.fori_loop(0, SL, acc_body, 0)

        pltpu.sync_copy(a0v, out0_hbm.at[pl.ds(cbase, C)])
        pltpu.sync_copy(a1v, out1_hbm.at[pl.ds(cbase, C)])


def _pack_tab(g, n):
    u = lax.bitcast_convert_type(g.astype(jnp.bfloat16), jnp.uint16)
    u = u.reshape(n // 64, 128)
    lo = u[:, 0::2].astype(jnp.uint32)
    hi = u[:, 1::2].astype(jnp.uint32)
    w = lax.bitcast_convert_type(lo | (hi << jnp.uint32(16)), jnp.int32)
    return w.reshape(n)


def kernel(coords, grid0, grid1, grid2, grid3, grid4, grid5, grid6, grid7):
    grids = (grid0, grid1, grid2, grid3, grid4, grid5, grid6, grid7)
    ct = coords.T
    x = ct[0]
    y = ct[1]
    tabs = [_pack_tab(g, r * r) for g, r in zip(grids, RESOLUTIONS)]
    small = jnp.concatenate(tabs[:N_SMALL], axis=0)
    o0, o1 = _grid_kernel(x, y, small, tabs[5], tabs[6], tabs[7])
    return jnp.stack([o0, o1], axis=1).astype(jnp.float16)


# speedup vs baseline: 8.0989x; 8.0989x over previous
import functools

import jax
import jax.numpy as jnp
from jax import lax
from jax.experimental import pallas as pl
from jax.experimental.pallas import tpu as pltpu
from jax.experimental.pallas import tpu_sc as plsc

RESOLUTIONS = (16, 32, 64, 128, 256, 512, 1024, 2048)
N_SMALL = 5
SMALL_OFF = (0, 256, 1280, 5376, 21760)
SMALL_TOT = 87296
BN = 524288
NC, NS = 2, 16
NW = NC * NS
NPT = BN // NW
C = 1024
NCH = NPT // C
SL = C // 16

_CLIP_HI = 1.0 - 1e-6


def _cell(xc, yc, r):
    rf = jnp.float32(r - 1)
    xs = xc * rf
    ys = yc * rf
    x0 = xs.astype(jnp.int32)
    y0 = ys.astype(jnp.int32)
    fx = xs - x0.astype(jnp.float32)
    fy = ys - y0.astype(jnp.float32)
    idx = y0 * r + x0
    return idx, fx, fy


def _f16_bits_to_f32(o, neg):
    n = o + jnp.int32(112 << 23)
    fs = plsc.bitcast(n + jnp.int32(1 << 23), jnp.float32) - jnp.float32(6.103515625e-05)
    f = jnp.where(o < (1 << 23), fs, plsc.bitcast(n, jnp.float32))
    return jnp.where(neg, -f, f)


def _f16_pair(w):
    o0 = lax.shift_left(w & 0x7FFF, 13)
    hi = lax.shift_right_logical(w, 16)
    o1 = lax.shift_left(hi & 0x7FFF, 13)
    v0 = _f16_bits_to_f32(o0, (w & 0x8000) != 0)
    v1 = _f16_bits_to_f32(o1, hi > 0x7FFF)
    return v0, v1


def _lerp_exact(w00, w10, w01, w11, fx, fy):
    a00, b00 = _f16_pair(w00)
    a10, b10 = _f16_pair(w10)
    a01, b01 = _f16_pair(w01)
    a11, b11 = _f16_pair(w11)
    a0 = a00 + (a10 - a00) * fx
    a1 = a01 + (a11 - a01) * fx
    b0 = b00 + (b10 - b00) * fx
    b1 = b01 + (b11 - b01) * fx
    return a0 + (a1 - a0) * fy, b0 + (b1 - b0) * fy


def _lerp_packed(w00, w10, w01, w11, fx, fy):
    c00 = plsc.bitcast(w00, jnp.bfloat16)
    c10 = plsc.bitcast(w10, jnp.bfloat16)
    c01 = plsc.bitcast(w01, jnp.bfloat16)
    c11 = plsc.bitcast(w11, jnp.bfloat16)
    fxd = plsc.pack(fx, fx, format=plsc.PackFormat.INTERLEAVED)
    fyd = plsc.pack(fy, fy, format=plsc.PackFormat.INTERLEAVED)
    l0 = c00 + (c10 - c00) * fxd
    l1 = c01 + (c11 - c01) * fxd
    lf = l0 + (l1 - l0) * fyd
    f0, f1 = plsc.unpack(lf, format=plsc.PackFormat.INTERLEAVED)
    return f0, f1


@functools.partial(
    pl.kernel,
    out_type=(jax.ShapeDtypeStruct((BN,), jnp.float32),
              jax.ShapeDtypeStruct((BN,), jnp.float32)),
    mesh=plsc.VectorSubcoreMesh(core_axis_name="c", subcore_axis_name="s",
                                num_cores=NC, num_subcores=NS),
    scratch_types=[
        pltpu.VMEM((SMALL_TOT,), jnp.int32),
        pltpu.VMEM((C,), jnp.float32),
        pltpu.VMEM((C,), jnp.float32),
        pltpu.VMEM((C,), jnp.float32),
        pltpu.VMEM((C,), jnp.float32),
        pltpu.VMEM((2 * C,), jnp.float32),
        pltpu.VMEM((2 * C,), jnp.float32),
        pltpu.VMEM((2 * C,), jnp.float32),
        pltpu.VMEM((4 * C,), jnp.int32),
        pltpu.VMEM((4 * C,), jnp.int32),
        pltpu.VMEM((4 * C,), jnp.int32),
        pltpu.VMEM((4 * C,), jnp.int32),
        pltpu.VMEM((4 * C,), jnp.int32),
        pltpu.VMEM((4 * C,), jnp.int32),
        pltpu.SemaphoreType.DMA,
        pltpu.SemaphoreType.DMA,
        pltpu.SemaphoreType.DMA,
    ],
    compiler_params=pltpu.CompilerParams(needs_layout_passes=False),
)
def _grid_kernel(x_hbm, y_hbm, small_hbm, t5_hbm, t6_hbm, t7_hbm,
                 out0_hbm, out1_hbm,
                 small_v, xv, yv, a0v, a1v, fw5, fw6, fw7,
                 idx5, idx6, idx7, rows5, rows6, rows7,
                 sem5, sem6, sem7):
    wid = lax.axis_index("s") * NC + lax.axis_index("c")
    base = wid * NPT
    pltpu.sync_copy(small_hbm, small_v)
    sems = (sem5, sem6, sem7)
    tabs = (t5_hbm, t6_hbm, t7_hbm)
    fws = (fw5, fw6, fw7)
    idxs = (idx5, idx6, idx7)
    rows = (rows5, rows6, rows7)
    def chunk_body(g, chunk_carry):
            cbase = base + g * C
            pltpu.sync_copy(x_hbm.at[pl.ds(cbase, C)], xv)
            pltpu.sync_copy(y_hbm.at[pl.ds(cbase, C)], yv)

            def idx_body(s, carry):
                o = s * 16
                xc = jnp.clip(xv[pl.ds(o, 16)], 0.0, _CLIP_HI)
                yc = jnp.clip(yv[pl.ds(o, 16)], 0.0, _CLIP_HI)
                for li, r in enumerate(RESOLUTIONS[N_SMALL:]):
                    idx, fx, fy = _cell(xc, yc, r)
                    iv = idxs[li]
                    iv[pl.ds(o, 16)] = idx
                    iv[pl.ds(C + o, 16)] = idx + 1
                    iv[pl.ds(2 * C + o, 16)] = idx + r
                    iv[pl.ds(3 * C + o, 16)] = idx + r + 1
                    fws[li][pl.ds(o, 16)] = fx
                    fws[li][pl.ds(C + o, 16)] = fy
                return carry

            lax.fori_loop(0, SL, idx_body, 0)

            copies = [
                pltpu.async_copy(tabs[li].at[idxs[li]], rows[li], sems[li])
                for li in range(3)
            ]

            def small_body(s, carry):
                o = s * 16
                xc = jnp.clip(xv[pl.ds(o, 16)], 0.0, _CLIP_HI)
                yc = jnp.clip(yv[pl.ds(o, 16)], 0.0, _CLIP_HI)
                a0 = jnp.zeros((16,), jnp.float32)
                a1 = jnp.zeros((16,), jnp.float32)
                for l in range(N_SMALL):
                    r = RESOLUTIONS[l]
                    idx, fx, fy = _cell(xc, yc, r)
                    b = idx + SMALL_OFF[l]
                    w00 = plsc.load_gather(small_v, [b])
                    w10 = plsc.load_gather(small_v, [b + 1])
                    w01 = plsc.load_gather(small_v, [b + r])
                    w11 = plsc.load_gather(small_v, [b + r + 1])
                    f0, f1 = _lerp_packed(w00, w10, w01, w11, fx, fy)
                    a0 = a0 + f0
                    a1 = a1 + f1
                a0v[pl.ds(o, 16)] = a0
                a1v[pl.ds(o, 16)] = a1
                return carry

            lax.fori_loop(0, SL, small_body, 0)

            for cp in copies:
                cp.wait()

            def acc_body(s, carry):
                o = s * 16
                s0 = jnp.zeros((16,), jnp.float32)
                s1 = jnp.zeros((16,), jnp.float32)
                for li in range(3):
                    rv = rows[li]
                    w00 = rv[pl.ds(o, 16)]
                    w10 = rv[pl.ds(C + o, 16)]
                    w01 = rv[pl.ds(2 * C + o, 16)]
                    w11 = rv[pl.ds(3 * C + o, 16)]
                    f0, f1 = _lerp_exact(w00, w10, w01, w11,
                                         fws[li][pl.ds(o, 16)],
                                         fws[li][pl.ds(C + o, 16)])
                    s0 = s0 + f0
                    s1 = s1 + f1
                a0v[pl.ds(o, 16)] = a0v[pl.ds(o, 16)] + s0
                a1v[pl.ds(o, 16)] = a1v[pl.ds(o, 16)] + s1
                return carry

            lax.fori_loop(0, SL, acc_body, 0)

            pltpu.sync_copy(a0v, out0_hbm.at[pl.ds(cbase, C)])
            pltpu.sync_copy(a1v, out1_hbm.at[pl.ds(cbase, C)])
            return chunk_carry

    lax.fori_loop(0, NCH, chunk_body, 0)


def _pack_tab(g):
    return lax.bitcast_convert_type(g.astype(jnp.bfloat16), jnp.int32)


def kernel(coords, grid0, grid1, grid2, grid3, grid4, grid5, grid6, grid7):
    grids = (grid0, grid1, grid2, grid3, grid4, grid5, grid6, grid7)
    ct = coords.T
    x = ct[0]
    y = ct[1]
    small = jnp.concatenate([_pack_tab(g) for g in grids[:N_SMALL]], axis=0)
    raw = [lax.bitcast_convert_type(g, jnp.int32) for g in grids[N_SMALL:]]
    o0, o1 = _grid_kernel(x, y, small, raw[0], raw[1], raw[2])
    return jnp.stack([o0, o1], axis=1).astype(jnp.float16)


# speedup vs baseline: 9.1946x; 1.1353x over previous
import functools

import jax
import jax.numpy as jnp
from jax import lax
from jax.experimental import pallas as pl
from jax.experimental.pallas import tpu as pltpu
from jax.experimental.pallas import tpu_sc as plsc

RESOLUTIONS = (16, 32, 64, 128, 256, 512, 1024, 2048)
N_SMALL = 5
SMALL_OFF = (0, 256, 1280, 5376, 21760)
SMALL_TOT = 87296
BN = 524288
NC, NS = 2, 16
NW = NC * NS
NPT = BN // NW
C = 512
NCH = NPT // C
SL = C // 16

_CLIP_HI = 1.0 - 1e-6


def _cell(xc, yc, r):
    rf = jnp.float32(r - 1)
    xs = xc * rf
    ys = yc * rf
    x0 = xs.astype(jnp.int32)
    y0 = ys.astype(jnp.int32)
    fx = xs - x0.astype(jnp.float32)
    fy = ys - y0.astype(jnp.float32)
    idx = y0 * r + x0
    return idx, fx, fy


def _f16_bits_to_f32(o, neg):
    n = o + jnp.int32(112 << 23)
    fs = plsc.bitcast(n + jnp.int32(1 << 23), jnp.float32) - jnp.float32(6.103515625e-05)
    f = jnp.where(o < (1 << 23), fs, plsc.bitcast(n, jnp.float32))
    return jnp.where(neg, -f, f)


def _f16_pair(w):
    o0 = lax.shift_left(w & 0x7FFF, 13)
    hi = lax.shift_right_logical(w, 16)
    o1 = lax.shift_left(hi & 0x7FFF, 13)
    v0 = _f16_bits_to_f32(o0, (w & 0x8000) != 0)
    v1 = _f16_bits_to_f32(o1, hi > 0x7FFF)
    return v0, v1


def _lerp_exact(w00, w10, w01, w11, fx, fy):
    a00, b00 = _f16_pair(w00)
    a10, b10 = _f16_pair(w10)
    a01, b01 = _f16_pair(w01)
    a11, b11 = _f16_pair(w11)
    a0 = a00 + (a10 - a00) * fx
    a1 = a01 + (a11 - a01) * fx
    b0 = b00 + (b10 - b00) * fx
    b1 = b01 + (b11 - b01) * fx
    return a0 + (a1 - a0) * fy, b0 + (b1 - b0) * fy


def _lerp_packed(w00, w10, w01, w11, fx, fy):
    c00 = plsc.bitcast(w00, jnp.bfloat16)
    c10 = plsc.bitcast(w10, jnp.bfloat16)
    c01 = plsc.bitcast(w01, jnp.bfloat16)
    c11 = plsc.bitcast(w11, jnp.bfloat16)
    fxd = plsc.pack(fx, fx, format=plsc.PackFormat.INTERLEAVED)
    fyd = plsc.pack(fy, fy, format=plsc.PackFormat.INTERLEAVED)
    l0 = c00 + (c10 - c00) * fxd
    l1 = c01 + (c11 - c01) * fxd
    lf = l0 + (l1 - l0) * fyd
    f0, f1 = plsc.unpack(lf, format=plsc.PackFormat.INTERLEAVED)
    return f0, f1


@functools.partial(
    pl.kernel,
    out_type=(jax.ShapeDtypeStruct((BN,), jnp.float32),
              jax.ShapeDtypeStruct((BN,), jnp.float32)),
    mesh=plsc.VectorSubcoreMesh(core_axis_name="c", subcore_axis_name="s",
                                num_cores=NC, num_subcores=NS),
    scratch_types=[
        pltpu.VMEM((SMALL_TOT,), jnp.int32),
        pltpu.VMEM((C,), jnp.float32),
        pltpu.VMEM((C,), jnp.float32),
        pltpu.VMEM((C,), jnp.float32),
        pltpu.VMEM((C,), jnp.float32),
        pltpu.VMEM((2 * C,), jnp.float32),
        pltpu.VMEM((2 * C,), jnp.float32),
        pltpu.VMEM((2 * C,), jnp.float32),
        pltpu.VMEM((2 * C,), jnp.float32),
        pltpu.VMEM((2 * C,), jnp.float32),
        pltpu.VMEM((2 * C,), jnp.float32),
        pltpu.VMEM((4 * C,), jnp.int32),
        pltpu.VMEM((4 * C,), jnp.int32),
        pltpu.VMEM((4 * C,), jnp.int32),
        pltpu.VMEM((4 * C,), jnp.int32),
        pltpu.VMEM((4 * C,), jnp.int32),
        pltpu.VMEM((4 * C,), jnp.int32),
        pltpu.VMEM((4 * C,), jnp.int32),
        pltpu.VMEM((4 * C,), jnp.int32),
        pltpu.VMEM((4 * C,), jnp.int32),
        pltpu.SemaphoreType.DMA,
        pltpu.SemaphoreType.DMA,
        pltpu.SemaphoreType.DMA,
    ],
    compiler_params=pltpu.CompilerParams(needs_layout_passes=False),
)
def _grid_kernel(x_hbm, y_hbm, small_hbm, t5_hbm, t6_hbm, t7_hbm,
                 out0_hbm, out1_hbm,
                 small_v, xv, yv, a0v, a1v,
                 fwA5, fwA6, fwA7, fwB5, fwB6, fwB7,
                 idx5, idx6, idx7,
                 rowsA5, rowsA6, rowsA7, rowsB5, rowsB6, rowsB7,
                 sem5, sem6, sem7):
    wid = lax.axis_index("s") * NC + lax.axis_index("c")
    base = wid * NPT
    pltpu.sync_copy(small_hbm, small_v)
    sems = (sem5, sem6, sem7)
    tabs = (t5_hbm, t6_hbm, t7_hbm)
    idxs = (idx5, idx6, idx7)
    fwsA = (fwA5, fwA6, fwA7)
    fwsB = (fwB5, fwB6, fwB7)
    rowsA = (rowsA5, rowsA6, rowsA7)
    rowsB = (rowsB5, rowsB6, rowsB7)

    def prep(gn, fws, rows):
        cbase = base + gn * C
        pltpu.sync_copy(x_hbm.at[pl.ds(cbase, C)], xv)
        pltpu.sync_copy(y_hbm.at[pl.ds(cbase, C)], yv)

        def idx_body(s, carry):
            o = s * 16
            xc = jnp.clip(xv[pl.ds(o, 16)], 0.0, _CLIP_HI)
            yc = jnp.clip(yv[pl.ds(o, 16)], 0.0, _CLIP_HI)
            for li, r in enumerate(RESOLUTIONS[N_SMALL:]):
                idx, fx, fy = _cell(xc, yc, r)
                iv = idxs[li]
                iv[pl.ds(o, 16)] = idx
                iv[pl.ds(C + o, 16)] = idx + 1
                iv[pl.ds(2 * C + o, 16)] = idx + r
                iv[pl.ds(3 * C + o, 16)] = idx + r + 1
                fws[li][pl.ds(o, 16)] = fx
                fws[li][pl.ds(C + o, 16)] = fy
            return carry

        lax.fori_loop(0, SL, idx_body, 0)
        return [pltpu.async_copy(tabs[li].at[idxs[li]], rows[li], sems[li])
                for li in range(3)]

    def small_pass():
        def small_body(s, carry):
            o = s * 16
            xc = jnp.clip(xv[pl.ds(o, 16)], 0.0, _CLIP_HI)
            yc = jnp.clip(yv[pl.ds(o, 16)], 0.0, _CLIP_HI)
            a0 = jnp.zeros((16,), jnp.float32)
            a1 = jnp.zeros((16,), jnp.float32)
            for l in range(N_SMALL):
                r = RESOLUTIONS[l]
                idx, fx, fy = _cell(xc, yc, r)
                b = idx + SMALL_OFF[l]
                w00 = plsc.load_gather(small_v, [b])
                w10 = plsc.load_gather(small_v, [b + 1])
                w01 = plsc.load_gather(small_v, [b + r])
                w11 = plsc.load_gather(small_v, [b + r + 1])
                f0, f1 = _lerp_packed(w00, w10, w01, w11, fx, fy)
                a0 = a0 + f0
                a1 = a1 + f1
            a0v[pl.ds(o, 16)] = a0
            a1v[pl.ds(o, 16)] = a1
            return carry

        lax.fori_loop(0, SL, small_body, 0)

    def acc_out(gn, fws, rows):
        def acc_body(s, carry):
            o = s * 16
            s0 = jnp.zeros((16,), jnp.float32)
            s1 = jnp.zeros((16,), jnp.float32)
            for li in range(3):
                rv = rows[li]
                w00 = rv[pl.ds(o, 16)]
                w10 = rv[pl.ds(C + o, 16)]
                w01 = rv[pl.ds(2 * C + o, 16)]
                w11 = rv[pl.ds(3 * C + o, 16)]
                f0, f1 = _lerp_exact(w00, w10, w01, w11,
                                     fws[li][pl.ds(o, 16)],
                                     fws[li][pl.ds(C + o, 16)])
                s0 = s0 + f0
                s1 = s1 + f1
            a0v[pl.ds(o, 16)] = a0v[pl.ds(o, 16)] + s0
            a1v[pl.ds(o, 16)] = a1v[pl.ds(o, 16)] + s1
            return carry

        lax.fori_loop(0, SL, acc_body, 0)
        cbase = base + gn * C
        pltpu.sync_copy(a0v, out0_hbm.at[pl.ds(cbase, C)])
        pltpu.sync_copy(a1v, out1_hbm.at[pl.ds(cbase, C)])

    prep(0, fwsA, rowsA)
    HALF = NCH // 2

    def wait3(rows):
        for li in range(3):
            pltpu.make_async_copy(tabs[li].at[idxs[li]], rows[li],
                                  sems[li]).wait()

    def pipe(gg, carry):
        g0 = gg * 2
        small_pass()
        wait3(rowsA)
        prep(g0 + 1, fwsB, rowsB)
        acc_out(g0, fwsA, rowsA)
        small_pass()
        wait3(rowsB)
        prep(g0 + 2, fwsA, rowsA)
        acc_out(g0 + 1, fwsB, rowsB)
        return carry

    lax.fori_loop(0, HALF - 1, pipe, 0)
    g_last = NCH - 2
    small_pass()
    wait3(rowsA)
    prep(g_last + 1, fwsB, rowsB)
    acc_out(g_last, fwsA, rowsA)
    small_pass()
    wait3(rowsB)
    acc_out(g_last + 1, fwsB, rowsB)


def _pack_tab(g):
    return lax.bitcast_convert_type(g.astype(jnp.bfloat16), jnp.int32)


def kernel(coords, grid0, grid1, grid2, grid3, grid4, grid5, grid6, grid7):
    grids = (grid0, grid1, grid2, grid3, grid4, grid5, grid6, grid7)
    ct = coords.T
    x = ct[0]
    y = ct[1]
    small = jnp.concatenate([_pack_tab(g) for g in grids[:N_SMALL]], axis=0)
    raw = [lax.bitcast_convert_type(g, jnp.int32) for g in grids[N_SMALL:]]
    o0, o1 = _grid_kernel(x, y, small, raw[0], raw[1], raw[2])
    return jnp.stack([o0, o1], axis=1).astype(jnp.float16)


# speedup vs baseline: 9.2249x; 1.0033x over previous
import functools

import jax
import jax.numpy as jnp
from jax import lax
from jax.experimental import pallas as pl
from jax.experimental.pallas import tpu as pltpu
from jax.experimental.pallas import tpu_sc as plsc

RESOLUTIONS = (16, 32, 64, 128, 256, 512, 1024, 2048)
N_SMALL = 5
SMALL_OFF = (0, 256, 1280, 5376, 21760)
SMALL_TOT = 87296
BN = 524288
NC, NS = 2, 16
NW = NC * NS
NPT = BN // NW
C = 512
NCH = NPT // C
SL = C // 16

_CLIP_HI = 1.0 - 1e-6


def _cell(xc, yc, r):
    rf = jnp.float32(r - 1)
    xs = xc * rf
    ys = yc * rf
    x0 = xs.astype(jnp.int32)
    y0 = ys.astype(jnp.int32)
    fx = xs - x0.astype(jnp.float32)
    fy = ys - y0.astype(jnp.float32)
    idx = y0 * r + x0
    return idx, fx, fy


def _f16_bits_to_f32(o, neg):
    n = o + jnp.int32(112 << 23)
    fs = plsc.bitcast(n + jnp.int32(1 << 23), jnp.float32) - jnp.float32(6.103515625e-05)
    f = jnp.where(o < (1 << 23), fs, plsc.bitcast(n, jnp.float32))
    return jnp.where(neg, -f, f)


def _f16_pair(w):
    o0 = lax.shift_left(w & 0x7FFF, 13)
    hi = lax.shift_right_logical(w, 16)
    o1 = lax.shift_left(hi & 0x7FFF, 13)
    v0 = _f16_bits_to_f32(o0, (w & 0x8000) != 0)
    v1 = _f16_bits_to_f32(o1, hi > 0x7FFF)
    return v0, v1


def _lerp_exact(w00, w10, w01, w11, fx, fy):
    a00, b00 = _f16_pair(w00)
    a10, b10 = _f16_pair(w10)
    a01, b01 = _f16_pair(w01)
    a11, b11 = _f16_pair(w11)
    a0 = a00 + (a10 - a00) * fx
    a1 = a01 + (a11 - a01) * fx
    b0 = b00 + (b10 - b00) * fx
    b1 = b01 + (b11 - b01) * fx
    return a0 + (a1 - a0) * fy, b0 + (b1 - b0) * fy


def _lerp_packed(w00, w10, w01, w11, fx, fy):
    c00 = plsc.bitcast(w00, jnp.bfloat16)
    c10 = plsc.bitcast(w10, jnp.bfloat16)
    c01 = plsc.bitcast(w01, jnp.bfloat16)
    c11 = plsc.bitcast(w11, jnp.bfloat16)
    fxd = plsc.pack(fx, fx, format=plsc.PackFormat.INTERLEAVED)
    fyd = plsc.pack(fy, fy, format=plsc.PackFormat.INTERLEAVED)
    l0 = c00 + (c10 - c00) * fxd
    l1 = c01 + (c11 - c01) * fxd
    lf = l0 + (l1 - l0) * fyd
    f0, f1 = plsc.unpack(lf, format=plsc.PackFormat.INTERLEAVED)
    return f0, f1


@functools.partial(
    pl.kernel,
    out_type=(jax.ShapeDtypeStruct((BN,), jnp.float32),
              jax.ShapeDtypeStruct((BN,), jnp.float32)),
    mesh=plsc.VectorSubcoreMesh(core_axis_name="c", subcore_axis_name="s",
                                num_cores=NC, num_subcores=NS),
    scratch_types=[
        pltpu.VMEM((SMALL_TOT,), jnp.int32),
        pltpu.VMEM((C,), jnp.float32),
        pltpu.VMEM((C,), jnp.float32),
        pltpu.VMEM((C,), jnp.float32),
        pltpu.VMEM((C,), jnp.float32),
        pltpu.VMEM((2 * C,), jnp.float32),
        pltpu.VMEM((2 * C,), jnp.float32),
        pltpu.VMEM((2 * C,), jnp.float32),
        pltpu.VMEM((2 * C,), jnp.float32),
        pltpu.VMEM((2 * C,), jnp.float32),
        pltpu.VMEM((2 * C,), jnp.float32),
        pltpu.VMEM((4 * C,), jnp.int32),
        pltpu.VMEM((4 * C,), jnp.int32),
        pltpu.VMEM((4 * C,), jnp.int32),
        pltpu.VMEM((4 * C,), jnp.int32),
        pltpu.VMEM((4 * C,), jnp.int32),
        pltpu.VMEM((4 * C,), jnp.int32),
        pltpu.VMEM((4 * C,), jnp.int32),
        pltpu.VMEM((4 * C,), jnp.int32),
        pltpu.VMEM((4 * C,), jnp.int32),
        pltpu.SemaphoreType.DMA,
        pltpu.SemaphoreType.DMA,
        pltpu.SemaphoreType.DMA,
    ],
    compiler_params=pltpu.CompilerParams(needs_layout_passes=False),
)
def _grid_kernel(x_hbm, y_hbm, small_hbm, t5_hbm, t6_hbm, t7_hbm,
                 out0_hbm, out1_hbm,
                 small_v, xv, yv, a0v, a1v,
                 fwA5, fwA6, fwA7, fwB5, fwB6, fwB7,
                 idx5, idx6, idx7,
                 rowsA5, rowsA6, rowsA7, rowsB5, rowsB6, rowsB7,
                 sem5, sem6, sem7):
    wid = lax.axis_index("s") * NC + lax.axis_index("c")
    base = wid * NPT
    pltpu.sync_copy(small_hbm, small_v)
    sems = (sem5, sem6, sem7)
    tabs = (t5_hbm, t6_hbm, t7_hbm)
    idxs = (idx5, idx6, idx7)
    fwsA = (fwA5, fwA6, fwA7)
    fwsB = (fwB5, fwB6, fwB7)
    rowsA = (rowsA5, rowsA6, rowsA7)
    rowsB = (rowsB5, rowsB6, rowsB7)

    def prep(gn, fws, rows):
        cbase = base + gn * C
        pltpu.sync_copy(x_hbm.at[pl.ds(cbase, C)], xv)
        pltpu.sync_copy(y_hbm.at[pl.ds(cbase, C)], yv)

        def idx_body(s, carry):
            o = s * 16
            xc = jnp.clip(xv[pl.ds(o, 16)], 0.0, _CLIP_HI)
            yc = jnp.clip(yv[pl.ds(o, 16)], 0.0, _CLIP_HI)
            for li, r in enumerate(RESOLUTIONS[N_SMALL:]):
                idx, fx, fy = _cell(xc, yc, r)
                iv = idxs[li]
                iv[pl.ds(o, 16)] = idx
                iv[pl.ds(C + o, 16)] = idx + 1
                iv[pl.ds(2 * C + o, 16)] = idx + r
                iv[pl.ds(3 * C + o, 16)] = idx + r + 1
                fws[li][pl.ds(o, 16)] = fx
                fws[li][pl.ds(C + o, 16)] = fy
            return carry

        lax.fori_loop(0, SL, idx_body, 0)
        return [pltpu.async_copy(tabs[li].at[idxs[li]], rows[li], sems[li])
                for li in range(3)]

    def small_pass():
        def small_body(s, carry):
            o = s * 16
            xc = jnp.clip(xv[pl.ds(o, 16)], 0.0, _CLIP_HI)
            yc = jnp.clip(yv[pl.ds(o, 16)], 0.0, _CLIP_HI)
            a0 = jnp.zeros((16,), jnp.float32)
            a1 = jnp.zeros((16,), jnp.float32)
            for l in range(N_SMALL):
                r = RESOLUTIONS[l]
                idx, fx, fy = _cell(xc, yc, r)
                b = idx + SMALL_OFF[l]
                w00 = plsc.load_gather(small_v, [b])
                w10 = plsc.load_gather(small_v, [b + 1])
                w01 = plsc.load_gather(small_v, [b + r])
                w11 = plsc.load_gather(small_v, [b + r + 1])
                f0, f1 = _lerp_packed(w00, w10, w01, w11, fx, fy)
                a0 = a0 + f0
                a1 = a1 + f1
            a0v[pl.ds(o, 16)] = a0
            a1v[pl.ds(o, 16)] = a1
            return carry

        lax.fori_loop(0, SL, small_body, 0)

    def acc_out(gn, fws, rows):
        def acc_body(s, carry):
            o = s * 16
            s0 = jnp.zeros((16,), jnp.float32)
            s1 = jnp.zeros((16,), jnp.float32)
            for li in range(3):
                rv = rows[li]
                w00 = rv[pl.ds(o, 16)]
                w10 = rv[pl.ds(C + o, 16)]
                w01 = rv[pl.ds(2 * C + o, 16)]
                w11 = rv[pl.ds(3 * C + o, 16)]
                lerp = _lerp_packed if li == 0 else _lerp_exact
                f0, f1 = lerp(w00, w10, w01, w11,
                              fws[li][pl.ds(o, 16)],
                              fws[li][pl.ds(C + o, 16)])
                s0 = s0 + f0
                s1 = s1 + f1
            a0v[pl.ds(o, 16)] = a0v[pl.ds(o, 16)] + s0
            a1v[pl.ds(o, 16)] = a1v[pl.ds(o, 16)] + s1
            return carry

        lax.fori_loop(0, SL, acc_body, 0)
        cbase = base + gn * C
        pltpu.sync_copy(a0v, out0_hbm.at[pl.ds(cbase, C)])
        pltpu.sync_copy(a1v, out1_hbm.at[pl.ds(cbase, C)])

    prep(0, fwsA, rowsA)
    HALF = NCH // 2

    def wait3(rows):
        for li in range(3):
            pltpu.make_async_copy(tabs[li].at[idxs[li]], rows[li],
                                  sems[li]).wait()

    def pipe(gg, carry):
        g0 = gg * 2
        small_pass()
        wait3(rowsA)
        prep(g0 + 1, fwsB, rowsB)
        acc_out(g0, fwsA, rowsA)
        small_pass()
        wait3(rowsB)
        prep(g0 + 2, fwsA, rowsA)
        acc_out(g0 + 1, fwsB, rowsB)
        return carry

    lax.fori_loop(0, HALF - 1, pipe, 0)
    g_last = NCH - 2
    small_pass()
    wait3(rowsA)
    prep(g_last + 1, fwsB, rowsB)
    acc_out(g_last, fwsA, rowsA)
    small_pass()
    wait3(rowsB)
    acc_out(g_last + 1, fwsB, rowsB)


def _pack_tab(g):
    return lax.bitcast_convert_type(g.astype(jnp.bfloat16), jnp.int32)


def kernel(coords, grid0, grid1, grid2, grid3, grid4, grid5, grid6, grid7):
    grids = (grid0, grid1, grid2, grid3, grid4, grid5, grid6, grid7)
    ct = coords.T
    x = ct[0]
    y = ct[1]
    small = jnp.concatenate([_pack_tab(g) for g in grids[:N_SMALL]], axis=0)
    t5 = _pack_tab(grids[5])
    raw6 = lax.bitcast_convert_type(grids[6], jnp.int32)
    raw7 = lax.bitcast_convert_type(grids[7], jnp.int32)
    o0, o1 = _grid_kernel(x, y, small, t5, raw6, raw7)
    return jnp.stack([o0, o1], axis=1).astype(jnp.float16)
